# Initial kernel scaffold; baseline (speedup 1.0000x reference)
import jax
import jax.numpy as jnp
from jax.experimental import pallas as pl


def kernel(x, inp_positions, out_positions, alpha, sigma, edge_list):
    raise NotImplementedError("write your pallas kernel here")


# speedup vs baseline: 29.3837x; 29.3837x over previous
import functools

import jax
import jax.numpy as jnp
from jax import lax
from jax.experimental import pallas as pl
from jax.experimental.pallas import tpu as pltpu
from jax.experimental.pallas import tpu_sc as plsc

NC = 2
NS = 16
NW = NC * NS
LANES = 16

BCH = 8
CHUNK = 1024
SUB = 128
KSUB = CHUNK // SUB


def _prep_body(p_ref, q_ref, sig_ref, al_ref, x_ref, stab_ref, dtab_ref):
    p = p_ref[...]
    q = q_ref[...]
    sig = sig_ref[...]
    inv = 1.0 / (sig * sig)
    xa = al_ref[...] * x_ref[...].T
    pn = jnp.sum(p * p, axis=1, keepdims=True)
    qn = jnp.sum(q * q, axis=1, keepdims=True)
    z1 = jnp.zeros_like(sig)
    z3 = jnp.zeros_like(p)
    stab_ref[...] = jnp.concatenate([2.0 * inv * p, -pn * inv, -inv, z3, xa],
                                    axis=1)
    dtab_ref[...] = jnp.concatenate(
        [q, z1, qn, jnp.zeros((q.shape[0], 11), jnp.float32)], axis=1)


def _edge_body(nchunk, rows_per_w, acc_per_tile,
               stab, dtab, sidx_hbm, didx_hbm, zeros_hbm, out_hbm,
               sidx, didx, srows, drows, contrib, acc, sem):
    c = lax.axis_index("c")
    s = lax.axis_index("s")
    wid = s * NC + c

    pltpu.sync_copy(zeros_hbm.at[pl.ds(s * acc_per_tile, acc_per_tile)],
                    acc.at[pl.ds(s * acc_per_tile, acc_per_tile)])
    plsc.subcore_barrier()

    base = wid * rows_per_w

    def chunk_body(k, carry):
        r0 = base + k * KSUB
        pltpu.sync_copy(sidx_hbm.at[pl.ds(r0, KSUB)], sidx)
        pltpu.sync_copy(didx_hbm.at[pl.ds(r0, KSUB)], didx)
        descs = []
        for j in range(KSUB):
            descs.append(pltpu.async_copy(
                stab.at[sidx.at[j]], srows.at[pl.ds(j * SUB, SUB)], sem))
            descs.append(pltpu.async_copy(
                dtab.at[didx.at[j]], drows.at[pl.ds(j * SUB, SUB)], sem))
        for d in descs:
            d.wait()

        def cbody(i, _):
            rows = lax.iota(jnp.int32, LANES) + i * LANES

            def gs(col):
                return plsc.load_gather(
                    srows, [rows, jnp.full((LANES,), col, jnp.int32)])

            def gd(col):
                return plsc.load_gather(
                    drows, [rows, jnp.full((LANES,), col, jnp.int32)])

            y = (gs(0) * gd(0) + gs(1) * gd(1) + gs(2) * gd(2)
                 + gs(3) + gs(4) * gd(4))
            w = jnp.exp(y)
            for b in range(BCH):
                plsc.store_scatter(
                    contrib, [rows, jnp.full((LANES,), b, jnp.int32)],
                    w * gs(8 + b))
            return 0

        lax.fori_loop(0, CHUNK // LANES, cbody, 0)

        for j in range(KSUB):
            pltpu.sync_copy(contrib.at[pl.ds(j * SUB, SUB)],
                            acc.at[didx.at[j]], add=True)
        return carry

    lax.fori_loop(0, nchunk, chunk_body, 0)

    plsc.subcore_barrier()
    pltpu.sync_copy(acc.at[pl.ds(s * acc_per_tile, acc_per_tile)],
                    out_hbm.at[c].at[pl.ds(s * acc_per_tile, acc_per_tile)])


def _merge_body(part_ref, out_ref):
    sblk = part_ref[0] + part_ref[1]
    out_ref[...] = sblk.T


def kernel(x, inp_positions, out_positions, alpha, sigma, edge_list):
    n_in = inp_positions.shape[0]
    n_out = out_positions.shape[0]
    e = edge_list.shape[1]
    assert n_out == n_in

    n_pad = -(-n_in // 128) * 128
    padn = n_pad - n_in
    p_pad = jnp.concatenate(
        [inp_positions, jnp.zeros((padn, 3), jnp.float32)], axis=0)
    q_pad = jnp.concatenate(
        [out_positions, jnp.zeros((padn, 3), jnp.float32)], axis=0)
    sig_pad = jnp.concatenate(
        [sigma.reshape(n_in, 1), jnp.ones((padn, 1), jnp.float32)], axis=0)
    al_pad = jnp.concatenate(
        [alpha.reshape(n_in, 1), jnp.zeros((padn, 1), jnp.float32)], axis=0)
    x_pad = jnp.concatenate(
        [x, jnp.zeros((BCH, padn), jnp.float32)], axis=1)

    cn = 2176
    assert n_pad % cn == 0
    grid = n_pad // cn
    stab, dtab = pl.pallas_call(
        _prep_body,
        grid=(grid,),
        in_specs=[
            pl.BlockSpec((cn, 3), lambda i: (i, 0)),
            pl.BlockSpec((cn, 3), lambda i: (i, 0)),
            pl.BlockSpec((cn, 1), lambda i: (i, 0)),
            pl.BlockSpec((cn, 1), lambda i: (i, 0)),
            pl.BlockSpec((BCH, cn), lambda i: (0, i)),
        ],
        out_specs=[
            pl.BlockSpec((cn, 16), lambda i: (i, 0)),
            pl.BlockSpec((cn, 16), lambda i: (i, 0)),
        ],
        out_shape=[
            jax.ShapeDtypeStruct((n_pad, 16), jnp.float32),
            jax.ShapeDtypeStruct((n_pad, 16), jnp.float32),
        ],
    )(p_pad, q_pad, sig_pad, al_pad, x_pad)

    ew = e // NW
    assert ew * NW == e and ew % 8 == 0
    nchunk = -(-ew // CHUNK)
    per_w = nchunk * CHUNK
    pad = per_w - ew
    dst = edge_list[0].reshape(NW, ew)
    src = edge_list[1].reshape(NW, ew)
    srcp = jnp.concatenate(
        [src, jnp.full((NW, pad), n_in, jnp.int32)], axis=1)
    dstp = jnp.concatenate(
        [dst, jnp.zeros((NW, pad), jnp.int32)], axis=1)
    rows_per_w = per_w // SUB
    sidx_hbm = srcp.reshape(NW * rows_per_w, SUB)
    didx_hbm = dstp.reshape(NW * rows_per_w, SUB)

    acc_n = n_pad
    acc_per_tile = acc_n // NS
    zeros_hbm = jnp.zeros((acc_n, BCH), jnp.float32)

    mesh = plsc.VectorSubcoreMesh(core_axis_name="c", subcore_axis_name="s",
                                  num_cores=NC, num_subcores=NS)
    part = pl.kernel(
        functools.partial(_edge_body, nchunk, rows_per_w, acc_per_tile),
        out_type=jax.ShapeDtypeStruct((NC, acc_n, BCH), jnp.float32),
        mesh=mesh,
        compiler_params=pltpu.CompilerParams(needs_layout_passes=False,
                                             use_tc_tiling_on_sc=False),
        scratch_types=[
            pltpu.VMEM((KSUB, SUB), jnp.int32),
            pltpu.VMEM((KSUB, SUB), jnp.int32),
            pltpu.VMEM((CHUNK, 16), jnp.float32),
            pltpu.VMEM((CHUNK, 16), jnp.float32),
            pltpu.VMEM((CHUNK, BCH), jnp.float32),
            pltpu.VMEM_SHARED((acc_n, BCH), jnp.float32),
            pltpu.SemaphoreType.DMA,
        ],
    )(stab, dtab, sidx_hbm, didx_hbm, zeros_hbm)

    cm = 2176
    out_pad = pl.pallas_call(
        _merge_body,
        grid=(n_pad // cm,),
        in_specs=[pl.BlockSpec((NC, cm, BCH), lambda i: (0, i, 0))],
        out_specs=pl.BlockSpec((BCH, cm), lambda i: (0, i)),
        out_shape=jax.ShapeDtypeStruct((BCH, n_pad), jnp.float32),
    )(part)
    return out_pad[:, :n_out]


# speedup vs baseline: 32.2389x; 1.0972x over previous
import functools

import jax
import jax.numpy as jnp
from jax import lax
from jax.experimental import pallas as pl
from jax.experimental.pallas import tpu as pltpu
from jax.experimental.pallas import tpu_sc as plsc

NC = 2
NS = 16
NW = NC * NS
LANES = 16

BCH = 8
CHUNK = 512
SUB = 128
KSUB = CHUNK // SUB


def _prep_body(p_ref, q_ref, sig_ref, al_ref, x_ref, stab_ref, dtab_ref):
    p = p_ref[...]
    q = q_ref[...]
    sig = sig_ref[...]
    inv = 1.0 / (sig * sig)
    xa = al_ref[...] * x_ref[...].T
    pn = jnp.sum(p * p, axis=1, keepdims=True)
    qn = jnp.sum(q * q, axis=1, keepdims=True)
    z1 = jnp.zeros_like(sig)
    z3 = jnp.zeros_like(p)
    stab_ref[...] = jnp.concatenate([2.0 * inv * p, -pn * inv, -inv, z3, xa],
                                    axis=1)
    dtab_ref[...] = jnp.concatenate(
        [q, z1, qn, jnp.zeros((q.shape[0], 11), jnp.float32)], axis=1)


def _edge_body(nchunk, rows_per_w, acc_per_tile,
               stab, dtab, sidx_hbm, didx_hbm, zeros_hbm, out_hbm,
               sidx0, didx0, srows0, drows0, contrib0, sdidx0,
               sidx1, didx1, srows1, drows1, contrib1, sdidx1,
               acc, gsem0, gsem1, ssem0, ssem1):
    c = lax.axis_index("c")
    s = lax.axis_index("s")
    wid = s * NC + c

    pltpu.sync_copy(zeros_hbm.at[pl.ds(s * acc_per_tile, acc_per_tile)],
                    acc.at[pl.ds(s * acc_per_tile, acc_per_tile)])
    plsc.subcore_barrier()

    base = wid * rows_per_w
    bufs = ((sidx0, didx0, srows0, drows0, contrib0, sdidx0, gsem0, ssem0),
            (sidx1, didx1, srows1, drows1, contrib1, sdidx1, gsem1, ssem1))

    def fire_gathers(k, b):
        sidx, didx, srows, drows = b[0], b[1], b[2], b[3]
        gsem = b[6]
        r0 = base + k * KSUB
        pltpu.sync_copy(sidx_hbm.at[pl.ds(r0, KSUB)], sidx)
        pltpu.sync_copy(didx_hbm.at[pl.ds(r0, KSUB)], didx)
        for j in range(KSUB):
            pltpu.async_copy(stab.at[sidx.at[j]],
                             srows.at[pl.ds(j * SUB, SUB)], gsem)
            pltpu.async_copy(dtab.at[didx.at[j]],
                             drows.at[pl.ds(j * SUB, SUB)], gsem)

    def drain_gathers(b):
        sidx, didx, srows, drows = b[0], b[1], b[2], b[3]
        gsem = b[6]
        for j in range(KSUB):
            pltpu.make_async_copy(stab.at[sidx.at[j]],
                                  srows.at[pl.ds(j * SUB, SUB)], gsem).wait()
            pltpu.make_async_copy(dtab.at[didx.at[j]],
                                  drows.at[pl.ds(j * SUB, SUB)], gsem).wait()

    def compute(b):
        srows, drows, contrib = b[2], b[3], b[4]

        def cbody(i, _):
            rows = lax.iota(jnp.int32, LANES) + i * LANES

            def gs(col):
                return plsc.load_gather(
                    srows, [rows, jnp.full((LANES,), col, jnp.int32)])

            def gd(col):
                return plsc.load_gather(
                    drows, [rows, jnp.full((LANES,), col, jnp.int32)])

            y = (gs(0) * gd(0) + gs(1) * gd(1) + gs(2) * gd(2)
                 + gs(3) + gs(4) * gd(4))
            w = jnp.exp(y)
            for b_ in range(BCH):
                plsc.store_scatter(
                    contrib, [rows, jnp.full((LANES,), b_, jnp.int32)],
                    w * gs(8 + b_))
            return 0

        lax.fori_loop(0, CHUNK // LANES, cbody, 0)

    def fire_scatters(b):
        contrib, sdidx, ssem = b[4], b[5], b[7]
        for j in range(KSUB):
            pltpu.async_copy(contrib.at[pl.ds(j * SUB, SUB)],
                             acc.at[sdidx.at[j]], ssem, add=True)

    def drain_scatters(b):
        contrib, sdidx, ssem = b[4], b[5], b[7]
        for j in range(KSUB):
            pltpu.make_async_copy(contrib.at[pl.ds(j * SUB, SUB)],
                                  acc.at[sdidx.at[j]], ssem).wait()

    def phase(t, k, this_b, other_b):
        @pl.when(k + 1 < nchunk)
        def _():
            fire_gathers(k + 1, other_b)
        drain_gathers(this_b)

        @pl.when(t >= 1)
        def _():
            drain_scatters(this_b)
        pltpu.sync_copy(didx_hbm.at[pl.ds(base + k * KSUB, KSUB)], this_b[5])
        compute(this_b)
        fire_scatters(this_b)

    fire_gathers(0, bufs[0])

    def body(t, carry):
        phase(t, 2 * t, bufs[0], bufs[1])
        phase(t, 2 * t + 1, bufs[1], bufs[0])
        return carry

    lax.fori_loop(0, nchunk // 2, body, 0)
    drain_scatters(bufs[0])
    drain_scatters(bufs[1])

    plsc.subcore_barrier()
    pltpu.sync_copy(acc.at[pl.ds(s * acc_per_tile, acc_per_tile)],
                    out_hbm.at[c].at[pl.ds(s * acc_per_tile, acc_per_tile)])


def _merge_body(part_ref, out_ref):
    sblk = part_ref[0] + part_ref[1]
    out_ref[...] = sblk.T


def kernel(x, inp_positions, out_positions, alpha, sigma, edge_list):
    n_in = inp_positions.shape[0]
    n_out = out_positions.shape[0]
    e = edge_list.shape[1]
    assert n_out == n_in

    n_pad = -(-n_in // 128) * 128
    padn = n_pad - n_in
    p_pad = jnp.concatenate(
        [inp_positions, jnp.zeros((padn, 3), jnp.float32)], axis=0)
    q_pad = jnp.concatenate(
        [out_positions, jnp.zeros((padn, 3), jnp.float32)], axis=0)
    sig_pad = jnp.concatenate(
        [sigma.reshape(n_in, 1), jnp.ones((padn, 1), jnp.float32)], axis=0)
    al_pad = jnp.concatenate(
        [alpha.reshape(n_in, 1), jnp.zeros((padn, 1), jnp.float32)], axis=0)
    x_pad = jnp.concatenate(
        [x, jnp.zeros((BCH, padn), jnp.float32)], axis=1)

    cn = 2176
    assert n_pad % cn == 0
    grid = n_pad // cn
    stab, dtab = pl.pallas_call(
        _prep_body,
        grid=(grid,),
        in_specs=[
            pl.BlockSpec((cn, 3), lambda i: (i, 0)),
            pl.BlockSpec((cn, 3), lambda i: (i, 0)),
            pl.BlockSpec((cn, 1), lambda i: (i, 0)),
            pl.BlockSpec((cn, 1), lambda i: (i, 0)),
            pl.BlockSpec((BCH, cn), lambda i: (0, i)),
        ],
        out_specs=[
            pl.BlockSpec((cn, 16), lambda i: (i, 0)),
            pl.BlockSpec((cn, 16), lambda i: (i, 0)),
        ],
        out_shape=[
            jax.ShapeDtypeStruct((n_pad, 16), jnp.float32),
            jax.ShapeDtypeStruct((n_pad, 16), jnp.float32),
        ],
    )(p_pad, q_pad, sig_pad, al_pad, x_pad)

    ew = e // NW
    assert ew * NW == e and ew % 8 == 0
    nchunk = -(-ew // CHUNK)
    nchunk += nchunk % 2
    per_w = nchunk * CHUNK
    pad = per_w - ew
    dst = edge_list[0].reshape(NW, ew)
    src = edge_list[1].reshape(NW, ew)
    srcp = jnp.concatenate(
        [src, jnp.full((NW, pad), n_in, jnp.int32)], axis=1)
    dstp = jnp.concatenate(
        [dst, jnp.zeros((NW, pad), jnp.int32)], axis=1)
    rows_per_w = per_w // SUB
    sidx_hbm = srcp.reshape(NW * rows_per_w, SUB)
    didx_hbm = dstp.reshape(NW * rows_per_w, SUB)

    acc_n = n_pad
    acc_per_tile = acc_n // NS
    zeros_hbm = jnp.zeros((acc_n, BCH), jnp.float32)

    mesh = plsc.VectorSubcoreMesh(core_axis_name="c", subcore_axis_name="s",
                                  num_cores=NC, num_subcores=NS)
    part = pl.kernel(
        functools.partial(_edge_body, nchunk, rows_per_w, acc_per_tile),
        out_type=jax.ShapeDtypeStruct((NC, acc_n, BCH), jnp.float32),
        mesh=mesh,
        compiler_params=pltpu.CompilerParams(needs_layout_passes=False,
                                             use_tc_tiling_on_sc=False),
        scratch_types=(
            [pltpu.VMEM((KSUB, SUB), jnp.int32),
             pltpu.VMEM((KSUB, SUB), jnp.int32),
             pltpu.VMEM((CHUNK, 16), jnp.float32),
             pltpu.VMEM((CHUNK, 16), jnp.float32),
             pltpu.VMEM((CHUNK, BCH), jnp.float32),
             pltpu.VMEM((KSUB, SUB), jnp.int32)] * 2
            + [pltpu.VMEM_SHARED((acc_n, BCH), jnp.float32),
               pltpu.SemaphoreType.DMA, pltpu.SemaphoreType.DMA,
               pltpu.SemaphoreType.DMA, pltpu.SemaphoreType.DMA]),
    )(stab, dtab, sidx_hbm, didx_hbm, zeros_hbm)

    cm = 2176
    out_pad = pl.pallas_call(
        _merge_body,
        grid=(n_pad // cm,),
        in_specs=[pl.BlockSpec((NC, cm, BCH), lambda i: (0, i, 0))],
        out_specs=pl.BlockSpec((BCH, cm), lambda i: (0, i)),
        out_shape=jax.ShapeDtypeStruct((BCH, n_pad), jnp.float32),
    )(part)
    return out_pad[:, :n_out]


# speedup vs baseline: 47.8269x; 1.4835x over previous
import functools

import jax
import jax.numpy as jnp
from jax import lax
from jax.experimental import pallas as pl
from jax.experimental.pallas import tpu as pltpu
from jax.experimental.pallas import tpu_sc as plsc

NC = 2
NS = 16
NW = NC * NS
LANES = 16

BCH = 8
CHUNK = 512
SUB = 128
KSUB = CHUNK // SUB


def _prep_body(p_ref, q_ref, sig_ref, al_ref, x_ref, stab_ref, dtab_ref):
    p = p_ref[...]
    q = q_ref[...]
    sig = sig_ref[...]
    inv = 1.0 / (sig * sig)
    xa = al_ref[...] * x_ref[...].T
    pn = jnp.sum(p * p, axis=1, keepdims=True)
    qn = jnp.sum(q * q, axis=1, keepdims=True)
    z1 = jnp.zeros_like(sig)
    z3 = jnp.zeros_like(p)
    stab_ref[...] = jnp.concatenate([2.0 * inv * p, -pn * inv, -inv, z3, xa],
                                    axis=1)
    dtab_ref[...] = jnp.concatenate(
        [q, z1, qn, jnp.zeros((q.shape[0], 11), jnp.float32)], axis=1)


def _edge_body(nchunk, rows_per_w, acc_per_tile,
               stab, dtab, sidx_hbm, didx_hbm, zeros_hbm, out_hbm,
               sidx0, didx0, srows0, drows0, contrib0, sdidx0,
               sidx1, didx1, srows1, drows1, contrib1, sdidx1,
               acc, gsem0, gsem1, ssem0, ssem1):
    c = lax.axis_index("c")
    s = lax.axis_index("s")
    wid = s * NC + c

    pltpu.sync_copy(zeros_hbm.at[pl.ds(s * acc_per_tile, acc_per_tile)],
                    acc.at[pl.ds(s * acc_per_tile, acc_per_tile)])
    plsc.subcore_barrier()

    base = wid * rows_per_w
    bufs = ((sidx0, didx0, srows0, drows0, contrib0, sdidx0, gsem0, ssem0),
            (sidx1, didx1, srows1, drows1, contrib1, sdidx1, gsem1, ssem1))

    def fire_gathers(k, b):
        sidx, didx, srows, drows = b[0], b[1], b[2], b[3]
        gsem = b[6]
        r0 = base + k * KSUB
        pltpu.sync_copy(sidx_hbm.at[pl.ds(r0, KSUB)], sidx)
        pltpu.sync_copy(didx_hbm.at[pl.ds(r0, KSUB)], didx)
        for j in range(KSUB):
            pltpu.async_copy(stab.at[sidx.at[j]],
                             srows.at[pl.ds(j * SUB, SUB)], gsem)
            pltpu.async_copy(dtab.at[didx.at[j]],
                             drows.at[pl.ds(j * SUB, SUB)], gsem)

    def drain_gathers(b):
        sidx, didx, srows, drows = b[0], b[1], b[2], b[3]
        gsem = b[6]
        for j in range(KSUB):
            pltpu.make_async_copy(stab.at[sidx.at[j]],
                                  srows.at[pl.ds(j * SUB, SUB)], gsem).wait()
            pltpu.make_async_copy(dtab.at[didx.at[j]],
                                  drows.at[pl.ds(j * SUB, SUB)], gsem).wait()

    def compute(b):
        srows, drows, contrib = b[2], b[3], b[4]

        @functools.partial(plsc.parallel_loop, 0, CHUNK // LANES, unroll=4)
        def cbody(i):
            rows = lax.iota(jnp.int32, LANES) + i * LANES

            def gs(col):
                return plsc.load_gather(
                    srows, [rows, jnp.full((LANES,), col, jnp.int32)])

            def gd(col):
                return plsc.load_gather(
                    drows, [rows, jnp.full((LANES,), col, jnp.int32)])

            y = (gs(0) * gd(0) + gs(1) * gd(1) + gs(2) * gd(2)
                 + gs(3) + gs(4) * gd(4))
            w = jnp.exp(y)
            for b_ in range(BCH):
                plsc.store_scatter(
                    contrib, [rows, jnp.full((LANES,), b_, jnp.int32)],
                    w * gs(8 + b_))

    def fire_scatters(b):
        contrib, sdidx, ssem = b[4], b[5], b[7]
        for j in range(KSUB):
            pltpu.async_copy(contrib.at[pl.ds(j * SUB, SUB)],
                             acc.at[sdidx.at[j]], ssem, add=True)

    def drain_scatters(b):
        contrib, sdidx, ssem = b[4], b[5], b[7]
        for j in range(KSUB):
            pltpu.make_async_copy(contrib.at[pl.ds(j * SUB, SUB)],
                                  acc.at[sdidx.at[j]], ssem).wait()

    def phase(t, k, this_b, other_b):
        @pl.when(k + 1 < nchunk)
        def _():
            fire_gathers(k + 1, other_b)
        drain_gathers(this_b)

        @pl.when(t >= 1)
        def _():
            drain_scatters(this_b)
        pltpu.sync_copy(didx_hbm.at[pl.ds(base + k * KSUB, KSUB)], this_b[5])
        compute(this_b)
        fire_scatters(this_b)

    fire_gathers(0, bufs[0])

    def body(t, carry):
        phase(t, 2 * t, bufs[0], bufs[1])
        phase(t, 2 * t + 1, bufs[1], bufs[0])
        return carry

    lax.fori_loop(0, nchunk // 2, body, 0)
    drain_scatters(bufs[0])
    drain_scatters(bufs[1])

    plsc.subcore_barrier()
    pltpu.sync_copy(acc.at[pl.ds(s * acc_per_tile, acc_per_tile)],
                    out_hbm.at[c].at[pl.ds(s * acc_per_tile, acc_per_tile)])


def _merge_body(part_ref, out_ref):
    sblk = part_ref[0] + part_ref[1]
    out_ref[...] = sblk.T


def kernel(x, inp_positions, out_positions, alpha, sigma, edge_list):
    n_in = inp_positions.shape[0]
    n_out = out_positions.shape[0]
    e = edge_list.shape[1]
    assert n_out == n_in

    n_pad = -(-n_in // 128) * 128
    padn = n_pad - n_in
    p_pad = jnp.concatenate(
        [inp_positions, jnp.zeros((padn, 3), jnp.float32)], axis=0)
    q_pad = jnp.concatenate(
        [out_positions, jnp.zeros((padn, 3), jnp.float32)], axis=0)
    sig_pad = jnp.concatenate(
        [sigma.reshape(n_in, 1), jnp.ones((padn, 1), jnp.float32)], axis=0)
    al_pad = jnp.concatenate(
        [alpha.reshape(n_in, 1), jnp.zeros((padn, 1), jnp.float32)], axis=0)
    x_pad = jnp.concatenate(
        [x, jnp.zeros((BCH, padn), jnp.float32)], axis=1)

    cn = 2176
    assert n_pad % cn == 0
    grid = n_pad // cn
    stab, dtab = pl.pallas_call(
        _prep_body,
        grid=(grid,),
        in_specs=[
            pl.BlockSpec((cn, 3), lambda i: (i, 0)),
            pl.BlockSpec((cn, 3), lambda i: (i, 0)),
            pl.BlockSpec((cn, 1), lambda i: (i, 0)),
            pl.BlockSpec((cn, 1), lambda i: (i, 0)),
            pl.BlockSpec((BCH, cn), lambda i: (0, i)),
        ],
        out_specs=[
            pl.BlockSpec((cn, 16), lambda i: (i, 0)),
            pl.BlockSpec((cn, 16), lambda i: (i, 0)),
        ],
        out_shape=[
            jax.ShapeDtypeStruct((n_pad, 16), jnp.float32),
            jax.ShapeDtypeStruct((n_pad, 16), jnp.float32),
        ],
    )(p_pad, q_pad, sig_pad, al_pad, x_pad)

    ew = e // NW
    assert ew * NW == e and ew % 8 == 0
    nchunk = -(-ew // CHUNK)
    nchunk += nchunk % 2
    per_w = nchunk * CHUNK
    pad = per_w - ew
    dst = edge_list[0].reshape(NW, ew)
    src = edge_list[1].reshape(NW, ew)
    srcp = jnp.concatenate(
        [src, jnp.full((NW, pad), n_in, jnp.int32)], axis=1)
    dstp = jnp.concatenate(
        [dst, jnp.zeros((NW, pad), jnp.int32)], axis=1)
    rows_per_w = per_w // SUB
    sidx_hbm = srcp.reshape(NW * rows_per_w, SUB)
    didx_hbm = dstp.reshape(NW * rows_per_w, SUB)

    acc_n = n_pad
    acc_per_tile = acc_n // NS
    zeros_hbm = jnp.zeros((acc_n, BCH), jnp.float32)

    mesh = plsc.VectorSubcoreMesh(core_axis_name="c", subcore_axis_name="s",
                                  num_cores=NC, num_subcores=NS)
    part = pl.kernel(
        functools.partial(_edge_body, nchunk, rows_per_w, acc_per_tile),
        out_type=jax.ShapeDtypeStruct((NC, acc_n, BCH), jnp.float32),
        mesh=mesh,
        compiler_params=pltpu.CompilerParams(needs_layout_passes=False,
                                             use_tc_tiling_on_sc=False),
        scratch_types=(
            [pltpu.VMEM((KSUB, SUB), jnp.int32),
             pltpu.VMEM((KSUB, SUB), jnp.int32),
             pltpu.VMEM((CHUNK, 16), jnp.float32),
             pltpu.VMEM((CHUNK, 16), jnp.float32),
             pltpu.VMEM((CHUNK, BCH), jnp.float32),
             pltpu.VMEM((KSUB, SUB), jnp.int32)] * 2
            + [pltpu.VMEM_SHARED((acc_n, BCH), jnp.float32),
               pltpu.SemaphoreType.DMA, pltpu.SemaphoreType.DMA,
               pltpu.SemaphoreType.DMA, pltpu.SemaphoreType.DMA]),
    )(stab, dtab, sidx_hbm, didx_hbm, zeros_hbm)

    cm = 2176
    out_pad = pl.pallas_call(
        _merge_body,
        grid=(n_pad // cm,),
        in_specs=[pl.BlockSpec((NC, cm, BCH), lambda i: (0, i, 0))],
        out_specs=pl.BlockSpec((BCH, cm), lambda i: (0, i)),
        out_shape=jax.ShapeDtypeStruct((BCH, n_pad), jnp.float32),
    )(part)
    return out_pad[:, :n_out]
